# baseline (speedup 1.0000x reference)
import functools

import jax
import jax.numpy as jnp
from jax import lax
from jax.experimental import pallas as pl
from jax.experimental.pallas import tpu as pltpu
from jax.experimental.pallas import tpu_sc as plsc

NUM_SEG = 256
STRENGTH = 0.01
E = 6400000
NC, NS, L = 2, 16, 16
NW = NC * NS
PER_W = E // NW
CHUNK = 10000
NCHUNK = PER_W // CHUNK
VPC = CHUNK // L
UNROLL = 5


def _sc_partials(x, idx):
    mesh = plsc.VectorSubcoreMesh(core_axis_name="c", subcore_axis_name="s")

    @functools.partial(
        pl.kernel,
        out_type=(
            jax.ShapeDtypeStruct((NW, NUM_SEG // 8, 8 * L), jnp.float32),
            jax.ShapeDtypeStruct((NW, NUM_SEG // 8, 8 * L), jnp.float32),
        ),
        mesh=mesh,
        compiler_params=pltpu.CompilerParams(needs_layout_passes=False),
        scratch_types=[
            pltpu.VMEM((CHUNK,), jnp.float32),
            pltpu.VMEM((CHUNK,), jnp.float32),
            pltpu.VMEM((CHUNK,), jnp.int32),
            pltpu.VMEM((CHUNK,), jnp.int32),
            pltpu.VMEM((NUM_SEG, L), jnp.float32),
            pltpu.VMEM((NUM_SEG, L), jnp.float32),
            pltpu.VMEM((NUM_SEG // 8, 8 * L), jnp.float32),
            pltpu.VMEM((NUM_SEG // 8, 8 * L), jnp.float32),
            pltpu.SemaphoreType.DMA((2,)),
            pltpu.SemaphoreType.DMA((2,)),
        ],
    )
    def k(x_hbm, idx_hbm, sums_hbm, cnts_hbm, x_buf0, x_buf1, idx_buf0,
          idx_buf1, acc, cnt, acc_st, cnt_st, sem_x, sem_i):
        x_bufs = (x_buf0, x_buf1)
        idx_bufs = (idx_buf0, idx_buf1)
        wid = lax.axis_index("s") * NC + lax.axis_index("c")
        base = pl.multiple_of(wid * PER_W, 8)

        def copies(c, b):
            off = pl.multiple_of(base + c * CHUNK, 8)
            return (
                pltpu.make_async_copy(
                    x_hbm.at[pl.ds(off, CHUNK)], x_bufs[b], sem_x.at[b]),
                pltpu.make_async_copy(
                    idx_hbm.at[pl.ds(off, CHUNK)], idx_bufs[b], sem_i.at[b]),
            )

        def zero_body(i, _):
            acc[i, :] = jnp.zeros((L,), jnp.float32)
            cnt[i, :] = jnp.zeros((L,), jnp.float32)
            return 0
        lax.fori_loop(0, NUM_SEG, zero_body, 0)

        lanes = lax.iota(jnp.int32, L)
        ones = jnp.full((L,), 1.0, jnp.float32)

        for b in range(2):
            for d in copies(b, b):
                d.start()

        for c in range(NCHUNK):
            b = c % 2
            for d in copies(c, b):
                d.wait()

            def body(i, _):
                ivs, xvs = [], []
                for u in range(UNROLL):
                    off = i * (L * UNROLL) + u * L
                    ivs.append(idx_bufs[b][pl.ds(off, L)])
                    xvs.append(x_bufs[b][pl.ds(off, L)])
                for u in range(UNROLL):
                    plsc.addupdate_scatter(acc, [ivs[u], lanes], xvs[u])
                    plsc.addupdate_scatter(cnt, [ivs[u], lanes], ones)
                return 0
            lax.fori_loop(0, VPC // UNROLL, body, 0)

            if c + 2 < NCHUNK:
                for d in copies(c + 2, b):
                    d.start()

        def restage(r, _):
            for j in range(8):
                acc_st[r, pl.ds(j * L, L)] = acc[r * 8 + j, :]
                cnt_st[r, pl.ds(j * L, L)] = cnt[r * 8 + j, :]
            return 0
        lax.fori_loop(0, NUM_SEG // 8, restage, 0)
        pltpu.sync_copy(acc_st, sums_hbm.at[wid])
        pltpu.sync_copy(cnt_st, cnts_hbm.at[wid])

    return k(x, idx)


def _loss_body(s_ref, c_ref, t_ref, o_ref):
    a_s = jnp.sum(s_ref[...], axis=0)
    a_c = jnp.sum(c_ref[...], axis=0)
    g = (lax.broadcasted_iota(jnp.int32, (8 * L, 8), 0) // L
         == lax.broadcasted_iota(jnp.int32, (8 * L, 8), 1)
         ).astype(jnp.float32)
    s = jnp.dot(a_s, g, preferred_element_type=jnp.float32)
    c = jnp.dot(a_c, g, preferred_element_type=jnp.float32)
    d = s / c - t_ref[...]
    loss = jnp.sum(d * d) * jnp.float32(STRENGTH / NUM_SEG)
    o_ref[...] = jnp.broadcast_to(loss, (1, 1))


def kernel(x, idx, target_mean_weights):
    sums_p, cnts_p = _sc_partials(x, idx)
    loss = pl.pallas_call(
        _loss_body,
        out_shape=jax.ShapeDtypeStruct((1, 1), jnp.float32),
    )(sums_p, cnts_p, target_mean_weights.reshape(NUM_SEG // 8, 8))
    return loss.reshape(())


# speedup vs baseline: 1.0224x; 1.0224x over previous
import functools

import jax
import jax.numpy as jnp
from jax import lax
from jax.experimental import pallas as pl
from jax.experimental.pallas import tpu as pltpu
from jax.experimental.pallas import tpu_sc as plsc

NUM_SEG = 256
STRENGTH = 0.01
E = 6400000
NC, NS, L = 2, 16, 16
NW = NC * NS
PER_W = E // NW
CHUNK = 10000
NCHUNK = PER_W // CHUNK
VPC = CHUNK // L
UNROLL = 5


def _sc_partials(x, idx):
    mesh = plsc.VectorSubcoreMesh(core_axis_name="c", subcore_axis_name="s")

    @functools.partial(
        pl.kernel,
        out_type=(
            jax.ShapeDtypeStruct((NW, NUM_SEG // 8, 8 * L), jnp.float32),
            jax.ShapeDtypeStruct((NW, NUM_SEG // 8, 8 * L), jnp.float32),
        ),
        mesh=mesh,
        compiler_params=pltpu.CompilerParams(needs_layout_passes=False),
        scratch_types=[
            pltpu.VMEM((CHUNK,), jnp.float32),
            pltpu.VMEM((CHUNK,), jnp.float32),
            pltpu.VMEM((CHUNK,), jnp.int32),
            pltpu.VMEM((CHUNK,), jnp.int32),
            pltpu.VMEM((NUM_SEG, L), jnp.float32),
            pltpu.VMEM((NUM_SEG, L), jnp.float32),
            pltpu.VMEM((NUM_SEG // 8, 8 * L), jnp.float32),
            pltpu.VMEM((NUM_SEG // 8, 8 * L), jnp.float32),
            pltpu.SemaphoreType.DMA((2,)),
            pltpu.SemaphoreType.DMA((2,)),
        ],
    )
    def k(x_hbm, idx_hbm, sums_hbm, cnts_hbm, x_buf0, x_buf1, idx_buf0,
          idx_buf1, acc, cnt, acc_st, cnt_st, sem_x, sem_i):
        x_bufs = (x_buf0, x_buf1)
        idx_bufs = (idx_buf0, idx_buf1)
        wid = lax.axis_index("s") * NC + lax.axis_index("c")
        base = pl.multiple_of(wid * PER_W, 8)

        def copies(c, b):
            off = pl.multiple_of(base + c * CHUNK, 8)
            return (
                pltpu.make_async_copy(
                    x_hbm.at[pl.ds(off, CHUNK)], x_bufs[b], sem_x.at[b]),
                pltpu.make_async_copy(
                    idx_hbm.at[pl.ds(off, CHUNK)], idx_bufs[b], sem_i.at[b]),
            )

        for b in range(2):
            for d in copies(b, b):
                d.start()

        zeros = jnp.zeros((L,), jnp.float32)

        def zero_body(i, _):
            for j in range(8):
                acc[i * 8 + j, :] = zeros
                cnt[i * 8 + j, :] = zeros
            return 0
        lax.fori_loop(0, NUM_SEG // 8, zero_body, 0)

        lanes = lax.iota(jnp.int32, L)
        ones = jnp.full((L,), 1.0, jnp.float32)

        for c in range(NCHUNK):
            b = c % 2
            for d in copies(c, b):
                d.wait()

            def body(i, _):
                ivs, xvs = [], []
                for u in range(UNROLL):
                    off = i * (L * UNROLL) + u * L
                    ivs.append(idx_bufs[b][pl.ds(off, L)])
                    xvs.append(x_bufs[b][pl.ds(off, L)])
                for u in range(UNROLL):
                    plsc.addupdate_scatter(acc, [ivs[u], lanes], xvs[u])
                    plsc.addupdate_scatter(cnt, [ivs[u], lanes], ones)
                return 0
            lax.fori_loop(0, VPC // UNROLL, body, 0)

            if c + 2 < NCHUNK:
                for d in copies(c + 2, b):
                    d.start()

        def restage(r, _):
            for j in range(8):
                acc_st[r, pl.ds(j * L, L)] = acc[r * 8 + j, :]
                cnt_st[r, pl.ds(j * L, L)] = cnt[r * 8 + j, :]
            return 0
        lax.fori_loop(0, NUM_SEG // 8, restage, 0)
        pltpu.sync_copy(acc_st, sums_hbm.at[wid])
        pltpu.sync_copy(cnt_st, cnts_hbm.at[wid])

    return k(x, idx)


def _loss_body(s_ref, c_ref, t_ref, o_ref):
    a_s = jnp.sum(s_ref[...], axis=0)
    a_c = jnp.sum(c_ref[...], axis=0)
    g = (lax.broadcasted_iota(jnp.int32, (8 * L, 8), 0) // L
         == lax.broadcasted_iota(jnp.int32, (8 * L, 8), 1)
         ).astype(jnp.float32)
    s = jnp.dot(a_s, g, preferred_element_type=jnp.float32)
    c = jnp.dot(a_c, g, preferred_element_type=jnp.float32)
    d = s / c - t_ref[...]
    loss = jnp.sum(d * d) * jnp.float32(STRENGTH / NUM_SEG)
    o_ref[...] = jnp.broadcast_to(loss, (1, 1))


def kernel(x, idx, target_mean_weights):
    sums_p, cnts_p = _sc_partials(x, idx)
    loss = pl.pallas_call(
        _loss_body,
        out_shape=jax.ShapeDtypeStruct((1, 1), jnp.float32),
    )(sums_p, cnts_p, target_mean_weights.reshape(NUM_SEG // 8, 8))
    return loss.reshape(())


# speedup vs baseline: 1.0245x; 1.0020x over previous
import functools

import jax
import jax.numpy as jnp
from jax import lax
from jax.experimental import pallas as pl
from jax.experimental.pallas import tpu as pltpu
from jax.experimental.pallas import tpu_sc as plsc

NUM_SEG = 256
STRENGTH = 0.01
E = 6400000
NC, NS, L = 2, 16, 16
NW = NC * NS
PER_W = E // NW
CHUNK = 10000
NCHUNK = PER_W // CHUNK
VPC = CHUNK // L
UNROLL = 5


def _sc_partials(x, idx):
    mesh = plsc.VectorSubcoreMesh(core_axis_name="c", subcore_axis_name="s")

    @functools.partial(
        pl.kernel,
        out_type=(
            jax.ShapeDtypeStruct((NW, NUM_SEG // 8, 8 * L), jnp.float32),
            jax.ShapeDtypeStruct((NW, NUM_SEG // 8, 8 * L), jnp.float32),
        ),
        mesh=mesh,
        compiler_params=pltpu.CompilerParams(needs_layout_passes=False),
        scratch_types=[
            pltpu.VMEM((CHUNK,), jnp.float32),
            pltpu.VMEM((CHUNK,), jnp.float32),
            pltpu.VMEM((CHUNK,), jnp.int32),
            pltpu.VMEM((CHUNK,), jnp.int32),
            pltpu.VMEM((NUM_SEG, L), jnp.float32),
            pltpu.VMEM((NUM_SEG, L), jnp.float32),
            pltpu.VMEM((NUM_SEG // 8, 8 * L), jnp.float32),
            pltpu.VMEM((NUM_SEG // 8, 8 * L), jnp.float32),
            pltpu.SemaphoreType.DMA((2,)),
            pltpu.SemaphoreType.DMA((2,)),
        ],
    )
    def k(x_hbm, idx_hbm, sums_hbm, cnts_hbm, x_buf0, x_buf1, idx_buf0,
          idx_buf1, acc, cnt, acc_st, cnt_st, sem_x, sem_i):
        x_bufs = (x_buf0, x_buf1)
        idx_bufs = (idx_buf0, idx_buf1)
        wid = lax.axis_index("s") * NC + lax.axis_index("c")
        base = pl.multiple_of(wid * PER_W, 8)

        def copies(c, b):
            off = pl.multiple_of(base + c * CHUNK, 8)
            return (
                pltpu.make_async_copy(
                    x_hbm.at[pl.ds(off, CHUNK)], x_bufs[b], sem_x.at[b]),
                pltpu.make_async_copy(
                    idx_hbm.at[pl.ds(off, CHUNK)], idx_bufs[b], sem_i.at[b]),
            )

        for b in range(2):
            for d in copies(b, b):
                d.start()

        zeros = jnp.zeros((L,), jnp.float32)

        def zero_body(i, _):
            for j in range(8):
                acc[i * 8 + j, :] = zeros
                cnt[i * 8 + j, :] = zeros
            return 0
        lax.fori_loop(0, NUM_SEG // 8, zero_body, 0)

        lanes = lax.iota(jnp.int32, L)
        ones = jnp.full((L,), 1.0, jnp.float32)

        for c in range(NCHUNK):
            b = c % 2
            for d in copies(c, b):
                d.wait()

            def body(i, _):
                ivs, xvs = [], []
                for u in range(UNROLL):
                    off = i * (L * UNROLL) + u * L
                    ivs.append(idx_bufs[b][pl.ds(off, L)])
                    xvs.append(x_bufs[b][pl.ds(off, L)])
                for u in range(UNROLL):
                    plsc.addupdate_scatter(acc, [ivs[u], lanes], xvs[u])
                    plsc.addupdate_scatter(cnt, [ivs[u], lanes], ones)
                return 0
            lax.fori_loop(0, VPC // UNROLL, body, 0)

            if c + 2 < NCHUNK:
                for d in copies(c + 2, b):
                    d.start()

        def restage_acc(r, _):
            for j in range(8):
                acc_st[r, pl.ds(j * L, L)] = acc[r * 8 + j, :]
            return 0
        lax.fori_loop(0, NUM_SEG // 8, restage_acc, 0)
        out_cp = pltpu.make_async_copy(acc_st, sums_hbm.at[wid],
                                       sem_x.at[0])
        out_cp.start()

        def restage_cnt(r, _):
            for j in range(8):
                cnt_st[r, pl.ds(j * L, L)] = cnt[r * 8 + j, :]
            return 0
        lax.fori_loop(0, NUM_SEG // 8, restage_cnt, 0)
        pltpu.sync_copy(cnt_st, cnts_hbm.at[wid])
        out_cp.wait()

    return k(x, idx)


def _loss_body(s_ref, c_ref, t_ref, o_ref):
    a_s = jnp.sum(s_ref[...], axis=0)
    a_c = jnp.sum(c_ref[...], axis=0)
    g = (lax.broadcasted_iota(jnp.int32, (8 * L, 8), 0) // L
         == lax.broadcasted_iota(jnp.int32, (8 * L, 8), 1)
         ).astype(jnp.float32)
    s = jnp.dot(a_s, g, preferred_element_type=jnp.float32)
    c = jnp.dot(a_c, g, preferred_element_type=jnp.float32)
    d = s / c - t_ref[...]
    loss = jnp.sum(d * d) * jnp.float32(STRENGTH / NUM_SEG)
    o_ref[...] = jnp.broadcast_to(loss, (1, 1))


def kernel(x, idx, target_mean_weights):
    sums_p, cnts_p = _sc_partials(x, idx)
    loss = pl.pallas_call(
        _loss_body,
        out_shape=jax.ShapeDtypeStruct((1, 1), jnp.float32),
    )(sums_p, cnts_p, target_mean_weights.reshape(NUM_SEG // 8, 8))
    return loss.reshape(())
